# Initial kernel scaffold; baseline (speedup 1.0000x reference)
import jax
import jax.numpy as jnp
from jax.experimental import pallas as pl


def kernel(edge_index, num_nodes, emb_table, W1_l, b1_l, W1_r, b1_r, W2_l, b2_l, W2_r, b2_r, Wc, bc):
    raise NotImplementedError("write your pallas kernel here")


# speedup vs baseline: 4.6745x; 4.6745x over previous
import functools

import jax
import jax.numpy as jnp
from jax import lax
from jax.experimental import pallas as pl
from jax.experimental.pallas import tpu as pltpu
from jax.experimental.pallas import tpu_sc as plsc

N = 10000
E = 320000
D_IN = 128
D_H = 256

NC = 2
NS = 16
CHUNK = 80
EPT = E // NS
NITER = EPT // CHUNK
NPT = 624
NREM = N - NS * NPT
EPW = E // (NC * NS)


def _seg_sum_call(xcat, src2, dst, dh, with_deg):
  mesh = plsc.VectorSubcoreMesh(core_axis_name="c", subcore_axis_name="s")

  agg_type = jax.ShapeDtypeStruct((NC, N, dh), jnp.float32)
  out_type = (agg_type, jax.ShapeDtypeStruct((NC, N, 16), jnp.float32)) \
      if with_deg else agg_type
  scratch = [
      pltpu.VMEM((CHUNK,), jnp.int32),
      pltpu.VMEM((CHUNK,), jnp.int32),
      pltpu.VMEM((CHUNK, dh), jnp.float32),
      pltpu.VMEM_SHARED((N, dh), jnp.float32),
      pltpu.SemaphoreType.DMA,
  ]
  if with_deg:
    scratch.append(pltpu.VMEM((CHUNK, 16), jnp.float32))
    scratch.append(pltpu.VMEM_SHARED((N, 16), jnp.float32))

  def body(xcat_ref, src2_ref, dst_ref, *rest):
    if with_deg:
      (agg_ref, deg_ref, s_idx, d_idx, rows, acc, sem, ones16, dacc) = rest
    else:
      (agg_ref, s_idx, d_idx, rows, acc, sem) = rest
    c = lax.axis_index("c")
    s = lax.axis_index("s")
    base = s * NPT

    def zrow(r, carry):
      for k in range(dh // 16):
        rows[r, pl.ds(k * 16, 16)] = jnp.zeros((16,), jnp.float32)
      return carry
    lax.fori_loop(0, CHUNK, zrow, 0)
    nfull = NPT // CHUNK
    rem = NPT - nfull * CHUNK
    for j in range(nfull):
      pltpu.sync_copy(rows, acc.at[pl.ds(base + j * CHUNK, CHUNK)])
    if rem:
      pltpu.sync_copy(rows.at[pl.ds(0, rem)],
                      acc.at[pl.ds(base + nfull * CHUNK, rem)])

    @pl.when(s == 0)
    def _():
      pltpu.sync_copy(rows.at[pl.ds(0, NREM)], acc.at[pl.ds(NS * NPT, NREM)])

    if with_deg:
      def zrow16(r, carry):
        ones16[r, pl.ds(0, 16)] = jnp.zeros((16,), jnp.float32)
        return carry
      lax.fori_loop(0, CHUNK, zrow16, 0)
      for j in range(nfull):
        pltpu.sync_copy(ones16, dacc.at[pl.ds(base + j * CHUNK, CHUNK)])
      if rem:
        pltpu.sync_copy(ones16.at[pl.ds(0, rem)],
                        dacc.at[pl.ds(base + nfull * CHUNK, rem)])

      @pl.when(s == 0)
      def _():
        pltpu.sync_copy(ones16.at[pl.ds(0, NREM)],
                        dacc.at[pl.ds(NS * NPT, NREM)])

      def orow16(r, carry):
        ones16[r, pl.ds(0, 16)] = jnp.ones((16,), jnp.float32)
        return carry
      lax.fori_loop(0, CHUNK, orow16, 0)

    plsc.subcore_barrier()

    ebase = s * EPT

    def step(i, carry):
      off = ebase + i * CHUNK
      pltpu.sync_copy(src2_ref.at[pl.ds(c * E + off, CHUNK)], s_idx)
      pltpu.sync_copy(dst_ref.at[pl.ds(off, CHUNK)], d_idx)
      pltpu.async_copy(xcat_ref.at[s_idx], rows, sem).wait()
      pltpu.sync_copy(rows, acc.at[d_idx], add=True)
      if with_deg:
        @pl.when(jnp.logical_or(jnp.logical_and(c == 0, i < NITER // 2),
                                jnp.logical_and(c == 1, i >= NITER // 2)))
        def _():
          pltpu.sync_copy(ones16, dacc.at[d_idx], add=True)
      return carry

    lax.fori_loop(0, NITER, step, 0)

    plsc.subcore_barrier()

    pltpu.sync_copy(acc.at[pl.ds(base, NPT)],
                    agg_ref.at[c, pl.ds(base, NPT)])

    @pl.when(s == 0)
    def _():
      pltpu.sync_copy(acc.at[pl.ds(NS * NPT, NREM)],
                      agg_ref.at[c, pl.ds(NS * NPT, NREM)])

    if with_deg:
      pltpu.sync_copy(dacc.at[pl.ds(base, NPT)],
                      deg_ref.at[c, pl.ds(base, NPT)])

      @pl.when(s == 0)
      def _():
        pltpu.sync_copy(dacc.at[pl.ds(NS * NPT, NREM)],
                        deg_ref.at[c, pl.ds(NS * NPT, NREM)])

  f = pl.kernel(body, out_type=out_type, mesh=mesh,
                scratch_types=tuple(scratch),
                compiler_params=pltpu.CompilerParams(
                    use_tc_tiling_on_sc=False))
  return f(xcat, src2, dst)


def _readout_call(s_vec, src, dst):
  mesh = plsc.VectorSubcoreMesh(core_axis_name="c", subcore_axis_name="s")

  def body(s_ref, src_ref, dst_ref, out_ref, s_buf, i_s, i_d, o_buf, sem):
    c = lax.axis_index("c")
    s = lax.axis_index("s")
    w = s * NC + c
    base = w * EPW
    pltpu.sync_copy(s_ref, s_buf)
    pltpu.sync_copy(src_ref.at[pl.ds(base, EPW)], i_s)
    pltpu.sync_copy(dst_ref.at[pl.ds(base, EPW)], i_d)

    def step(j, carry):
      ia = i_s[pl.ds(j * 16, 16)]
      ib = i_d[pl.ds(j * 16, 16)]
      va = plsc.load_gather(s_buf, [ia])
      vb = plsc.load_gather(s_buf, [ib])
      o_buf[pl.ds(j * 16, 16)] = va + vb
      return carry

    lax.fori_loop(0, EPW // 16, step, 0)
    pltpu.sync_copy(o_buf, out_ref.at[pl.ds(base, EPW)])

  f = pl.kernel(body,
                out_type=jax.ShapeDtypeStruct((E,), jnp.float32),
                mesh=mesh,
                scratch_types=(
                    pltpu.VMEM((N,), jnp.float32),
                    pltpu.VMEM((EPW,), jnp.int32),
                    pltpu.VMEM((EPW,), jnp.int32),
                    pltpu.VMEM((EPW,), jnp.float32),
                    pltpu.SemaphoreType.DMA,
                ),
                compiler_params=pltpu.CompilerParams(
                    needs_layout_passes=False))
  return f(s_vec, src, dst)


BN = 400


def _tc1_body(agg_ref, deg_ref, emb_ref, wl_ref, wr_ref, bl_ref, br_ref,
              h_ref, inv_ref):
  deg = deg_ref[0][:, 0] + deg_ref[1][:, 0]
  inv = 1.0 / jnp.maximum(deg, 1.0)
  mean = jnp.concatenate([agg_ref[0], agg_ref[1]], axis=-1) * inv[:, None]
  h = (jnp.dot(mean, wl_ref[...], preferred_element_type=jnp.float32)
       + jnp.dot(emb_ref[...], wr_ref[...], preferred_element_type=jnp.float32)
       + (bl_ref[...] + br_ref[...])[None, :])
  h = jnp.maximum(h, 0.0)
  h_ref[0] = h[:, :128]
  h_ref[1] = h[:, 128:]
  inv_ref[pl.program_id(0)] = inv


def _tc2_body(agg_ref, inv_ref, h1_ref, wl_ref, wr_ref, bl_ref, br_ref,
              wc_ref, bc_ref, s_ref):
  inv = inv_ref[pl.program_id(0)]
  mean = jnp.concatenate([agg_ref[0], agg_ref[1]], axis=-1) * inv[:, None]
  h1 = jnp.concatenate([h1_ref[0], h1_ref[1]], axis=-1)
  h = (jnp.dot(mean, wl_ref[...], preferred_element_type=jnp.float32)
       + jnp.dot(h1, wr_ref[...], preferred_element_type=jnp.float32)
       + (bl_ref[...] + br_ref[...])[None, :])
  h = jnp.maximum(h, 0.0)
  s = jnp.sum(h * wc_ref[...][None, :], axis=1)
  s_ref[pl.program_id(0)] = s + bc_ref[0] * 0.5


def kernel(edge_index, num_nodes, emb_table, W1_l, b1_l, W1_r, b1_r,
           W2_l, b2_l, W2_r, b2_r, Wc, bc):
  src = edge_index[0]
  dst = edge_index[1]
  src2 = jnp.concatenate([src, src + N])

  x0 = lax.slice(emb_table, (0, 0), (N, D_IN))
  xcat1 = jnp.concatenate([x0[:, :64], x0[:, 64:]], axis=0)

  agg1, deg16 = _seg_sum_call(xcat1, src2, dst, D_IN // 2, True)

  grid = (N // BN,)
  h1, inv = pl.pallas_call(
      _tc1_body,
      grid=grid,
      in_specs=[
          pl.BlockSpec((NC, BN, 64), lambda i: (0, i, 0)),
          pl.BlockSpec((NC, BN, 16), lambda i: (0, i, 0)),
          pl.BlockSpec((BN, D_IN), lambda i: (i, 0)),
          pl.BlockSpec((D_IN, D_H), lambda i: (0, 0)),
          pl.BlockSpec((D_IN, D_H), lambda i: (0, 0)),
          pl.BlockSpec((D_H,), lambda i: (0,)),
          pl.BlockSpec((D_H,), lambda i: (0,)),
      ],
      out_specs=[
          pl.BlockSpec((NC, BN, 128), lambda i: (0, i, 0)),
          pl.BlockSpec((N // BN, BN), lambda i: (0, 0)),
      ],
      out_shape=[
          jax.ShapeDtypeStruct((NC, N, 128), jnp.float32),
          jax.ShapeDtypeStruct((N // BN, BN), jnp.float32),
      ],
  )(agg1, deg16, emb_table, W1_l, W1_r, b1_l, b1_r)

  xcat2 = jnp.reshape(h1, (NC * N, 128))
  agg2 = _seg_sum_call(xcat2, src2, dst, D_H // 2, False)

  s_vec = pl.pallas_call(
      _tc2_body,
      grid=grid,
      in_specs=[
          pl.BlockSpec((NC, BN, 128), lambda i: (0, i, 0)),
          pl.BlockSpec((N // BN, BN), lambda i: (0, 0)),
          pl.BlockSpec((NC, BN, 128), lambda i: (0, i, 0)),
          pl.BlockSpec((D_H, D_H), lambda i: (0, 0)),
          pl.BlockSpec((D_H, D_H), lambda i: (0, 0)),
          pl.BlockSpec((D_H,), lambda i: (0,)),
          pl.BlockSpec((D_H,), lambda i: (0,)),
          pl.BlockSpec((D_H,), lambda i: (0,)),
          pl.BlockSpec((1,), lambda i: (0,)),
      ],
      out_specs=pl.BlockSpec((N // BN, BN), lambda i: (0, 0)),
      out_shape=jax.ShapeDtypeStruct((N // BN, BN), jnp.float32),
  )(agg2, inv, h1, W2_l, W2_r, b2_l, b2_r, Wc[:, 0], bc)

  return _readout_call(jnp.reshape(s_vec, (N,)), src, dst)


# speedup vs baseline: 10.9595x; 2.3446x over previous
import functools

import jax
import jax.numpy as jnp
from jax import lax
from jax.experimental import pallas as pl
from jax.experimental.pallas import tpu as pltpu
from jax.experimental.pallas import tpu_sc as plsc

N = 10000
E = 320000
D_IN = 128
D_H = 256

NC = 2
NS = 16
CHUNK = 80
EPT = E // NS
NITER = EPT // CHUNK
NPT = 624
NREM = N - NS * NPT
EPW = E // (NC * NS)


def _seg_sum_call(xcat, src2, dst, dh, with_deg):
  mesh = plsc.VectorSubcoreMesh(core_axis_name="c", subcore_axis_name="s")

  agg_type = jax.ShapeDtypeStruct((NC, N, dh), jnp.float32)
  out_type = (agg_type, jax.ShapeDtypeStruct((NC, N, 16), jnp.float32)) \
      if with_deg else agg_type
  scratch = [
      pltpu.VMEM((CHUNK,), jnp.int32),
      pltpu.VMEM((CHUNK,), jnp.int32),
      pltpu.VMEM((CHUNK,), jnp.int32),
      pltpu.VMEM((CHUNK,), jnp.int32),
      pltpu.VMEM((CHUNK, dh), jnp.float32),
      pltpu.VMEM((CHUNK, dh), jnp.float32),
      pltpu.VMEM_SHARED((N, dh), jnp.float32),
      pltpu.SemaphoreType.DMA,
      pltpu.SemaphoreType.DMA,
      pltpu.SemaphoreType.DMA,
      pltpu.SemaphoreType.DMA,
      pltpu.SemaphoreType.DMA,
      pltpu.SemaphoreType.DMA,
      pltpu.SemaphoreType.DMA,
      pltpu.SemaphoreType.DMA,
  ]
  if with_deg:
    scratch.append(pltpu.VMEM((CHUNK, 16), jnp.float32))
    scratch.append(pltpu.VMEM_SHARED((N, 16), jnp.float32))

  def body(xcat_ref, src2_ref, dst_ref, *rest):
    if with_deg:
      (agg_ref, deg_ref, s0, s1, d0, d1, rows0, rows1, acc,
       gs0, gs1, ss0, ss1, is0, is1, id0, id1, ones16, dacc) = rest
    else:
      (agg_ref, s0, s1, d0, d1, rows0, rows1, acc,
       gs0, gs1, ss0, ss1, is0, is1, id0, id1) = rest
    c = lax.axis_index("c")
    s = lax.axis_index("s")
    base = s * NPT
    rows_b = (rows0, rows1)
    s_b = (s0, s1)
    d_b = (d0, d1)
    gs_b = (gs0, gs1)
    ss_b = (ss0, ss1)
    is_b = (is0, is1)
    id_b = (id0, id1)

    def sidx_start(i, b):
      pltpu.async_copy(src2_ref.at[pl.ds(c * E + s * EPT + i * CHUNK, CHUNK)],
                       s_b[b], is_b[b])

    def sidx_wait(b):
      pltpu.make_async_copy(src2_ref.at[pl.ds(0, CHUNK)],
                            s_b[b], is_b[b]).wait()

    def gather_start(b):
      pltpu.async_copy(xcat_ref.at[s_b[b]], rows_b[b], gs_b[b])

    def gather_wait(b):
      pltpu.make_async_copy(xcat_ref.at[s_b[b]], rows_b[b], gs_b[b]).wait()

    def didx_start(i, b):
      pltpu.async_copy(dst_ref.at[pl.ds(s * EPT + i * CHUNK, CHUNK)],
                       d_b[b], id_b[b])

    def didx_wait(b):
      pltpu.make_async_copy(dst_ref.at[pl.ds(0, CHUNK)],
                            d_b[b], id_b[b]).wait()

    def scatter_start(b):
      pltpu.async_copy(rows_b[b], acc.at[d_b[b]], ss_b[b], add=True)

    def scatter_wait(b):
      pltpu.make_async_copy(rows_b[b], acc.at[d_b[b]], ss_b[b]).wait()


    def zrow(r, carry):
      for k in range(dh // 16):
        rows0[r, pl.ds(k * 16, 16)] = jnp.zeros((16,), jnp.float32)
      return carry
    lax.fori_loop(0, CHUNK, zrow, 0)
    nfull = NPT // CHUNK
    rem = NPT - nfull * CHUNK
    for j in range(nfull):
      pltpu.sync_copy(rows0, acc.at[pl.ds(base + j * CHUNK, CHUNK)])
    if rem:
      pltpu.sync_copy(rows0.at[pl.ds(0, rem)],
                      acc.at[pl.ds(base + nfull * CHUNK, rem)])

    @pl.when(s == 0)
    def _():
      pltpu.sync_copy(rows0.at[pl.ds(0, NREM)], acc.at[pl.ds(NS * NPT, NREM)])

    if with_deg:
      def zrow16(r, carry):
        ones16[r, pl.ds(0, 16)] = jnp.zeros((16,), jnp.float32)
        return carry
      lax.fori_loop(0, CHUNK, zrow16, 0)
      for j in range(nfull):
        pltpu.sync_copy(ones16, dacc.at[pl.ds(base + j * CHUNK, CHUNK)])
      if rem:
        pltpu.sync_copy(ones16.at[pl.ds(0, rem)],
                        dacc.at[pl.ds(base + nfull * CHUNK, rem)])

      @pl.when(s == 0)
      def _():
        pltpu.sync_copy(ones16.at[pl.ds(0, NREM)],
                        dacc.at[pl.ds(NS * NPT, NREM)])

      def orow16(r, carry):
        ones16[r, pl.ds(0, 16)] = jnp.ones((16,), jnp.float32)
        return carry
      lax.fori_loop(0, CHUNK, orow16, 0)

    plsc.subcore_barrier()

    def deg_scatter(i, b):
      if with_deg:
        @pl.when(jnp.logical_or(jnp.logical_and(c == 0, i < NITER // 2),
                                jnp.logical_and(c == 1, i >= NITER // 2)))
        def _():
          pltpu.sync_copy(ones16, dacc.at[d_b[b]], add=True)

    sidx_start(0, 0)
    sidx_start(1, 1)
    didx_start(0, 0)
    didx_start(1, 1)
    sidx_wait(0)
    gather_start(0)

    def step(i2, carry):
      ie = 2 * i2
      io = 2 * i2 + 1
      last = NITER // 2 - 1

      @pl.when(i2 > 0)
      def _():
        scatter_wait(1)
        didx_start(io, 1)
      sidx_wait(1)
      gather_start(1)
      gather_wait(0)

      @pl.when(i2 < last)
      def _():
        sidx_start(ie + 2, 0)
      didx_wait(0)
      scatter_start(0)
      deg_scatter(ie, 0)

      scatter_wait(0)

      @pl.when(i2 < last)
      def _():
        didx_start(ie + 2, 0)
        sidx_wait(0)
        gather_start(0)
      gather_wait(1)

      @pl.when(i2 < last)
      def _():
        sidx_start(io + 2, 1)
      didx_wait(1)
      scatter_start(1)
      deg_scatter(io, 1)
      return carry

    lax.fori_loop(0, NITER // 2, step, 0)
    scatter_wait(1)

    plsc.subcore_barrier()

    pltpu.sync_copy(acc.at[pl.ds(base, NPT)],
                    agg_ref.at[c, pl.ds(base, NPT)])

    @pl.when(s == 0)
    def _():
      pltpu.sync_copy(acc.at[pl.ds(NS * NPT, NREM)],
                      agg_ref.at[c, pl.ds(NS * NPT, NREM)])

    if with_deg:
      pltpu.sync_copy(dacc.at[pl.ds(base, NPT)],
                      deg_ref.at[c, pl.ds(base, NPT)])

      @pl.when(s == 0)
      def _():
        pltpu.sync_copy(dacc.at[pl.ds(NS * NPT, NREM)],
                        deg_ref.at[c, pl.ds(NS * NPT, NREM)])

  f = pl.kernel(body, out_type=out_type, mesh=mesh,
                scratch_types=tuple(scratch),
                compiler_params=pltpu.CompilerParams(
                    use_tc_tiling_on_sc=False))
  return f(xcat, src2, dst)


def _readout_call(s_vec, src, dst):
  mesh = plsc.VectorSubcoreMesh(core_axis_name="c", subcore_axis_name="s")

  def body(s_ref, src_ref, dst_ref, out_ref, s_buf, i_s, i_d, o_buf, sem):
    c = lax.axis_index("c")
    s = lax.axis_index("s")
    w = s * NC + c
    base = w * EPW
    pltpu.sync_copy(s_ref, s_buf)
    pltpu.sync_copy(src_ref.at[pl.ds(base, EPW)], i_s)
    pltpu.sync_copy(dst_ref.at[pl.ds(base, EPW)], i_d)

    def step(j, carry):
      ia = i_s[pl.ds(j * 16, 16)]
      ib = i_d[pl.ds(j * 16, 16)]
      va = plsc.load_gather(s_buf, [ia])
      vb = plsc.load_gather(s_buf, [ib])
      o_buf[pl.ds(j * 16, 16)] = va + vb
      return carry

    lax.fori_loop(0, EPW // 16, step, 0)
    pltpu.sync_copy(o_buf, out_ref.at[pl.ds(base, EPW)])

  f = pl.kernel(body,
                out_type=jax.ShapeDtypeStruct((E,), jnp.float32),
                mesh=mesh,
                scratch_types=(
                    pltpu.VMEM((N,), jnp.float32),
                    pltpu.VMEM((EPW,), jnp.int32),
                    pltpu.VMEM((EPW,), jnp.int32),
                    pltpu.VMEM((EPW,), jnp.float32),
                    pltpu.SemaphoreType.DMA,
                ),
                compiler_params=pltpu.CompilerParams(
                    needs_layout_passes=False))
  return f(s_vec, src, dst)


BN = 400


def _tc1_body(agg_ref, deg_ref, emb_ref, wl_ref, wr_ref, bl_ref, br_ref,
              h_ref, inv_ref):
  deg = deg_ref[0][:, 0] + deg_ref[1][:, 0]
  inv = 1.0 / jnp.maximum(deg, 1.0)
  mean = jnp.concatenate([agg_ref[0], agg_ref[1]], axis=-1) * inv[:, None]
  h = (jnp.dot(mean, wl_ref[...], preferred_element_type=jnp.float32)
       + jnp.dot(emb_ref[...], wr_ref[...], preferred_element_type=jnp.float32)
       + (bl_ref[...] + br_ref[...])[None, :])
  h = jnp.maximum(h, 0.0)
  h_ref[0] = h[:, :128]
  h_ref[1] = h[:, 128:]
  inv_ref[pl.program_id(0)] = inv


def _tc2_body(agg_ref, inv_ref, h1_ref, wl_ref, wr_ref, bl_ref, br_ref,
              wc_ref, bc_ref, s_ref):
  inv = inv_ref[pl.program_id(0)]
  mean = jnp.concatenate([agg_ref[0], agg_ref[1]], axis=-1) * inv[:, None]
  h1 = jnp.concatenate([h1_ref[0], h1_ref[1]], axis=-1)
  h = (jnp.dot(mean, wl_ref[...], preferred_element_type=jnp.float32)
       + jnp.dot(h1, wr_ref[...], preferred_element_type=jnp.float32)
       + (bl_ref[...] + br_ref[...])[None, :])
  h = jnp.maximum(h, 0.0)
  s = jnp.sum(h * wc_ref[...][None, :], axis=1)
  s_ref[pl.program_id(0)] = s + bc_ref[0] * 0.5


def kernel(edge_index, num_nodes, emb_table, W1_l, b1_l, W1_r, b1_r,
           W2_l, b2_l, W2_r, b2_r, Wc, bc):
  src = edge_index[0]
  dst = edge_index[1]
  src2 = jnp.concatenate([src, src + N])

  x0 = lax.slice(emb_table, (0, 0), (N, D_IN))
  xcat1 = jnp.concatenate([x0[:, :64], x0[:, 64:]], axis=0)

  agg1, deg16 = _seg_sum_call(xcat1, src2, dst, D_IN // 2, True)

  grid = (N // BN,)
  h1, inv = pl.pallas_call(
      _tc1_body,
      grid=grid,
      in_specs=[
          pl.BlockSpec((NC, BN, 64), lambda i: (0, i, 0)),
          pl.BlockSpec((NC, BN, 16), lambda i: (0, i, 0)),
          pl.BlockSpec((BN, D_IN), lambda i: (i, 0)),
          pl.BlockSpec((D_IN, D_H), lambda i: (0, 0)),
          pl.BlockSpec((D_IN, D_H), lambda i: (0, 0)),
          pl.BlockSpec((D_H,), lambda i: (0,)),
          pl.BlockSpec((D_H,), lambda i: (0,)),
      ],
      out_specs=[
          pl.BlockSpec((NC, BN, 128), lambda i: (0, i, 0)),
          pl.BlockSpec((N // BN, BN), lambda i: (0, 0)),
      ],
      out_shape=[
          jax.ShapeDtypeStruct((NC, N, 128), jnp.float32),
          jax.ShapeDtypeStruct((N // BN, BN), jnp.float32),
      ],
  )(agg1, deg16, emb_table, W1_l, W1_r, b1_l, b1_r)

  xcat2 = jnp.reshape(h1, (NC * N, 128))
  agg2 = _seg_sum_call(xcat2, src2, dst, D_H // 2, False)

  s_vec = pl.pallas_call(
      _tc2_body,
      grid=grid,
      in_specs=[
          pl.BlockSpec((NC, BN, 128), lambda i: (0, i, 0)),
          pl.BlockSpec((N // BN, BN), lambda i: (0, 0)),
          pl.BlockSpec((NC, BN, 128), lambda i: (0, i, 0)),
          pl.BlockSpec((D_H, D_H), lambda i: (0, 0)),
          pl.BlockSpec((D_H, D_H), lambda i: (0, 0)),
          pl.BlockSpec((D_H,), lambda i: (0,)),
          pl.BlockSpec((D_H,), lambda i: (0,)),
          pl.BlockSpec((D_H,), lambda i: (0,)),
          pl.BlockSpec((1,), lambda i: (0,)),
      ],
      out_specs=pl.BlockSpec((N // BN, BN), lambda i: (0, 0)),
      out_shape=jax.ShapeDtypeStruct((N // BN, BN), jnp.float32),
  )(agg2, inv, h1, W2_l, W2_r, b2_l, b2_r, Wc[:, 0], bc)

  return _readout_call(jnp.reshape(s_vec, (N,)), src, dst)
